# baseline (speedup 1.0000x reference)
import functools

import jax
import jax.numpy as jnp
from jax import lax
from jax.experimental import pallas as pl
from jax.experimental.pallas import tpu as pltpu
from jax.experimental.pallas import tpu_sc as plsc

NC = 2
NS = 16
NW = NC * NS
L = 16

H = 128
CH = 128
EPS = 1e-12
HJ = H // L


def _rsqrt_newton(v):
    bits = lax.bitcast_convert_type(v, jnp.int32)
    y = lax.bitcast_convert_type(
        jnp.full((L,), 0x5F3759DF, jnp.int32) - lax.shift_right_arithmetic(bits, 1),
        jnp.float32)
    half = jnp.float32(0.5) * v
    for _ in range(3):
        y = y * (jnp.float32(1.5) - half * y * y)
    return y


def _allreduce_sum(v):
    lane = lax.iota(jnp.int32, L)
    dnums = lax.GatherDimensionNumbers(
        offset_dims=(), collapsed_slice_dims=(0,), start_index_map=(0,))
    for sh in (8, 4, 2, 1):
        v = v + lax.gather(v, (lane ^ sh)[:, None], dimension_numbers=dnums,
                           slice_sizes=(1,), unique_indices=True,
                           indices_are_sorted=False,
                           mode=lax.GatherScatterMode.PROMISE_IN_BOUNDS)
    return v


def _sc_body(n_per_w, n_chunks, seq, ids_hbm, table_hbm, pos_hbm, gamma_hbm,
             beta_hbm, out_hbm, idx_v, rows_v, pos_v, g_v, b_v, sem):
    cid = lax.axis_index("c")
    sid = lax.axis_index("s")
    wid = sid * NC + cid
    base = wid * n_per_w
    s_base = lax.rem(base, seq)

    pltpu.sync_copy(ids_hbm.at[wid], idx_v)
    pltpu.sync_copy(gamma_hbm, g_v)
    pltpu.sync_copy(beta_hbm, b_v)

    gs = [g_v[pl.ds(L * j, L)] for j in range(HJ)]
    bs = [b_v[pl.ds(L * j, L)] for j in range(HJ)]

    for ci in range(n_chunks):
        row0 = base + ci * CH
        pltpu.async_copy(table_hbm.at[idx_v.at[ci]], rows_v, sem).wait()
        pltpu.sync_copy(pos_hbm.at[pl.ds(s_base + ci * CH, CH)], pos_v)

        def row_body(r, carry):
            xs = [rows_v[r, pl.ds(L * j, L)] + pos_v[r, pl.ds(L * j, L)]
                  for j in range(HJ)]
            t01 = xs[0] + xs[1]
            t23 = xs[2] + xs[3]
            t45 = xs[4] + xs[5]
            t67 = xs[6] + xs[7]
            total = (t01 + t23) + (t45 + t67)
            mean = _allreduce_sum(total) * jnp.float32(1.0 / H)
            cs = [x - mean for x in xs]
            sq = [c * c for c in cs]
            q01 = sq[0] + sq[1]
            q23 = sq[2] + sq[3]
            q45 = sq[4] + sq[5]
            q67 = sq[6] + sq[7]
            qtot = (q01 + q23) + (q45 + q67)
            var = _allreduce_sum(qtot) * jnp.float32(1.0 / H)
            rstd = _rsqrt_newton(var + jnp.float32(EPS))
            for j in range(HJ):
                rows_v[r, pl.ds(L * j, L)] = cs[j] * rstd * gs[j] + bs[j]
            return carry

        lax.fori_loop(0, CH, row_body, 0, unroll=2)

        pltpu.sync_copy(rows_v, out_hbm.at[pl.ds(row0, CH)])


def _build_call(n, seq):
    n_per_w = n // NW
    n_chunks = n_per_w // CH
    mesh = plsc.VectorSubcoreMesh(core_axis_name="c", subcore_axis_name="s")
    return pl.kernel(
        functools.partial(_sc_body, n_per_w, n_chunks, seq),
        out_type=jax.ShapeDtypeStruct((n, H), jnp.float32),
        mesh=mesh,
        scratch_types=[
            pltpu.VMEM((n_chunks, CH), jnp.int32),
            pltpu.VMEM((CH, H), jnp.float32),
            pltpu.VMEM((CH, H), jnp.float32),
            pltpu.VMEM((H,), jnp.float32),
            pltpu.VMEM((H,), jnp.float32),
            pltpu.SemaphoreType.DMA,
        ],
    )


@jax.jit
def kernel(input_ids, word_table, pos_table, gamma, beta):
    b, s = input_ids.shape
    n = b * s
    ids = input_ids.reshape(NW, (n // NW) // CH, CH).astype(jnp.int32)
    call = _build_call(n, s)
    out = call(ids, word_table, pos_table, gamma, beta)
    return out.reshape(b, s, H)


# speedup vs baseline: 1.7083x; 1.7083x over previous
import functools

import jax
import jax.numpy as jnp
from jax import lax
from jax.experimental import pallas as pl
from jax.experimental.pallas import tpu as pltpu
from jax.experimental.pallas import tpu_sc as plsc

NC = 2
NS = 16
NW = NC * NS
L = 16

H = 128
CH = 128
EPS = 1e-12
HJ = H // L


def _rsqrt_newton(v):
    bits = lax.bitcast_convert_type(v, jnp.int32)
    y = lax.bitcast_convert_type(
        jnp.full((L,), 0x5F3759DF, jnp.int32) - lax.shift_right_arithmetic(bits, 1),
        jnp.float32)
    half = jnp.float32(0.5) * v
    for _ in range(3):
        y = y * (jnp.float32(1.5) - half * y * y)
    return y


def _allreduce_sum(v):
    lane = lax.iota(jnp.int32, L)
    dnums = lax.GatherDimensionNumbers(
        offset_dims=(), collapsed_slice_dims=(0,), start_index_map=(0,))
    for sh in (8, 4, 2, 1):
        v = v + lax.gather(v, (lane ^ sh)[:, None], dimension_numbers=dnums,
                           slice_sizes=(1,), unique_indices=True,
                           indices_are_sorted=False,
                           mode=lax.GatherScatterMode.PROMISE_IN_BOUNDS)
    return v


def _sc_body(n_per_w, n_chunks, seq, ids_hbm, table_hbm, pos_hbm, gamma_hbm,
             beta_hbm, out_hbm, idx_v, rows_v, pos_v, g_v, b_v,
             gsems, psems, osems):
    cid = lax.axis_index("c")
    sid = lax.axis_index("s")
    wid = sid * NC + cid
    base = wid * n_per_w
    s_base = lax.rem(base, seq)

    pltpu.sync_copy(ids_hbm.at[wid], idx_v)
    pltpu.sync_copy(gamma_hbm, g_v)
    pltpu.sync_copy(beta_hbm, b_v)

    gs = [g_v[pl.ds(L * j, L)] for j in range(HJ)]
    bs = [b_v[pl.ds(L * j, L)] for j in range(HJ)]

    def start_in(ci):
        slot = ci & 1
        g = pltpu.async_copy(table_hbm.at[idx_v.at[ci]], rows_v.at[slot],
                             gsems[slot])
        p = pltpu.async_copy(pos_hbm.at[pl.ds(s_base + ci * CH, CH)],
                             pos_v.at[slot], psems[slot])
        return g, p

    inflight = {0: start_in(0)}
    out_h = {}
    for ci in range(n_chunks):
        slot = ci & 1
        if ci + 1 < n_chunks:
            nxt = (ci + 1) & 1
            if nxt in out_h:
                out_h.pop(nxt).wait()
            inflight[ci + 1] = start_in(ci + 1)
        g, p = inflight.pop(ci)
        g.wait()
        p.wait()

        @plsc.parallel_loop(0, CH, step=1, unroll=4)
        def _(r):
            xs = [rows_v[slot, r, pl.ds(L * j, L)] +
                  pos_v[slot, r, pl.ds(L * j, L)] for j in range(HJ)]
            t01 = xs[0] + xs[1]
            t23 = xs[2] + xs[3]
            t45 = xs[4] + xs[5]
            t67 = xs[6] + xs[7]
            total = (t01 + t23) + (t45 + t67)
            sq = [x * x for x in xs]
            q01 = sq[0] + sq[1]
            q23 = sq[2] + sq[3]
            q45 = sq[4] + sq[5]
            q67 = sq[6] + sq[7]
            qtot = (q01 + q23) + (q45 + q67)
            mean = _allreduce_sum(total) * jnp.float32(1.0 / H)
            ex2 = _allreduce_sum(qtot) * jnp.float32(1.0 / H)
            var = ex2 - mean * mean
            rstd = _rsqrt_newton(var + jnp.float32(EPS))
            for j in range(HJ):
                m1 = rstd * gs[j]
                m2 = bs[j] - mean * m1
                rows_v[slot, r, pl.ds(L * j, L)] = xs[j] * m1 + m2

        out_h[slot] = pltpu.async_copy(
            rows_v.at[slot], out_hbm.at[pl.ds(base + ci * CH, CH)], osems[slot])
    for h in out_h.values():
        h.wait()


def _build_call(n, seq):
    n_per_w = n // NW
    n_chunks = n_per_w // CH
    mesh = plsc.VectorSubcoreMesh(core_axis_name="c", subcore_axis_name="s")
    return pl.kernel(
        functools.partial(_sc_body, n_per_w, n_chunks, seq),
        out_type=jax.ShapeDtypeStruct((n, H), jnp.float32),
        mesh=mesh,
        scratch_types=[
            pltpu.VMEM((n_chunks, CH), jnp.int32),
            pltpu.VMEM((2, CH, H), jnp.float32),
            pltpu.VMEM((2, CH, H), jnp.float32),
            pltpu.VMEM((H,), jnp.float32),
            pltpu.VMEM((H,), jnp.float32),
            [pltpu.SemaphoreType.DMA] * 2,
            [pltpu.SemaphoreType.DMA] * 2,
            [pltpu.SemaphoreType.DMA] * 2,
        ],
    )


@jax.jit
def kernel(input_ids, word_table, pos_table, gamma, beta):
    b, s = input_ids.shape
    n = b * s
    ids = input_ids.reshape(NW, (n // NW) // CH, CH).astype(jnp.int32)
    call = _build_call(n, s)
    out = call(ids, word_table, pos_table, gamma, beta)
    return out.reshape(b, s, H)


# speedup vs baseline: 1.9021x; 1.1135x over previous
import functools

import jax
import jax.numpy as jnp
from jax import lax
from jax.experimental import pallas as pl
from jax.experimental.pallas import tpu as pltpu
from jax.experimental.pallas import tpu_sc as plsc

NC = 2
NS = 16
NW = NC * NS
L = 16

H = 128
CH = 128
EPS = 1e-12
HJ = H // L


def _rsqrt_newton(v):
    bits = lax.bitcast_convert_type(v, jnp.int32)
    y = lax.bitcast_convert_type(
        jnp.full((L,), 0x5F3759DF, jnp.int32) - lax.shift_right_arithmetic(bits, 1),
        jnp.float32)
    half = jnp.float32(0.5) * v
    for _ in range(2):
        y = y * (jnp.float32(1.5) - half * y * y)
    return y


def _allreduce_sum(v):
    lane = lax.iota(jnp.int32, L)
    dnums = lax.GatherDimensionNumbers(
        offset_dims=(), collapsed_slice_dims=(0,), start_index_map=(0,))
    for sh in (8, 4, 2, 1):
        v = v + lax.gather(v, (lane ^ sh)[:, None], dimension_numbers=dnums,
                           slice_sizes=(1,), unique_indices=True,
                           indices_are_sorted=False,
                           mode=lax.GatherScatterMode.PROMISE_IN_BOUNDS)
    return v


def _sc_body(batch, s_per_w, seq, ids_hbm, table_hbm, pos_hbm, gamma_hbm,
             beta_hbm, out_hbm, idx_v, rows_v, pos_v, g_v, b_v,
             isem, psem, gsems, osems):
    cid = lax.axis_index("c")
    sid = lax.axis_index("s")
    wid = sid * NC + cid
    s0 = wid * s_per_w
    n_sub = s_per_w // CH
    chunks = [(b, h) for b in range(batch) for h in range(n_sub)]

    ih = [pltpu.async_copy(ids_hbm.at[b, wid], idx_v.at[b], isem)
          for b in range(batch)]
    ph = pltpu.async_copy(pos_hbm.at[pl.ds(s0, s_per_w)], pos_v, psem)
    pltpu.sync_copy(gamma_hbm, g_v)
    pltpu.sync_copy(beta_hbm, b_v)
    for h in ih:
        h.wait()

    gs = [g_v[pl.ds(L * j, L)] for j in range(HJ)]
    bs = [b_v[pl.ds(L * j, L)] for j in range(HJ)]

    def start_gather(ci):
        b, h = chunks[ci]
        slot = ci & 1
        return pltpu.async_copy(table_hbm.at[idx_v.at[b, h]],
                                rows_v.at[slot], gsems[slot])

    inflight = {0: start_gather(0)}
    out_h = {}
    ph.wait()
    for ci in range(len(chunks)):
        b, h = chunks[ci]
        slot = ci & 1
        if ci + 1 < len(chunks):
            nxt = (ci + 1) & 1
            if nxt in out_h:
                out_h.pop(nxt).wait()
            inflight[ci + 1] = start_gather(ci + 1)
        inflight.pop(ci).wait()

        @plsc.parallel_loop(0, CH, step=1, unroll=4)
        def _(r):
            pr = h * CH + r
            xs = [rows_v[slot, r, pl.ds(L * j, L)] +
                  pos_v[pr, pl.ds(L * j, L)] for j in range(HJ)]
            t01 = xs[0] + xs[1]
            t23 = xs[2] + xs[3]
            t45 = xs[4] + xs[5]
            t67 = xs[6] + xs[7]
            total = (t01 + t23) + (t45 + t67)
            sq = [x * x for x in xs]
            q01 = sq[0] + sq[1]
            q23 = sq[2] + sq[3]
            q45 = sq[4] + sq[5]
            q67 = sq[6] + sq[7]
            qtot = (q01 + q23) + (q45 + q67)
            mean = _allreduce_sum(total) * jnp.float32(1.0 / H)
            ex2 = _allreduce_sum(qtot) * jnp.float32(1.0 / H)
            var = ex2 - mean * mean
            rstd = _rsqrt_newton(var + jnp.float32(EPS))
            for j in range(HJ):
                rows_v[slot, r, pl.ds(L * j, L)] = \
                    ((xs[j] - mean) * rstd) * gs[j] + bs[j]

        row0 = b * seq + s0 + h * CH
        out_h[slot] = pltpu.async_copy(
            rows_v.at[slot], out_hbm.at[pl.ds(row0, CH)], osems[slot])
    for hdl in out_h.values():
        hdl.wait()


def _build_call(batch, seq):
    s_per_w = seq // NW
    n_sub = s_per_w // CH
    mesh = plsc.VectorSubcoreMesh(core_axis_name="c", subcore_axis_name="s")
    return pl.kernel(
        functools.partial(_sc_body, batch, s_per_w, seq),
        out_type=jax.ShapeDtypeStruct((batch * seq, H), jnp.float32),
        mesh=mesh,
        scratch_types=[
            pltpu.VMEM((batch, n_sub, CH), jnp.int32),
            pltpu.VMEM((2, CH, H), jnp.float32),
            pltpu.VMEM((s_per_w, H), jnp.float32),
            pltpu.VMEM((H,), jnp.float32),
            pltpu.VMEM((H,), jnp.float32),
            pltpu.SemaphoreType.DMA,
            pltpu.SemaphoreType.DMA,
            [pltpu.SemaphoreType.DMA] * 2,
            [pltpu.SemaphoreType.DMA] * 2,
        ],
    )


@jax.jit
def kernel(input_ids, word_table, pos_table, gamma, beta):
    b, s = input_ids.shape
    s_per_w = s // NW
    ids = input_ids.reshape(b, NW, s_per_w // CH, CH).astype(jnp.int32)
    call = _build_call(b, s)
    out = call(ids, word_table, pos_table, gamma, beta)
    return out.reshape(b, s, H)
